# baseline (speedup 1.0000x reference)
import functools

import jax
import jax.numpy as jnp
from jax import lax
from jax.experimental import pallas as pl
from jax.experimental.pallas import tpu as pltpu
from jax.experimental.pallas import tpu_sc as plsc

_B, _C, _H, _W = 4, 4, 384, 384
_N = _H * _W
_K = _N * 10 // 100
_ROWS = _N // 128
_NSLICES = _B * _C
_CH = 16384
_NCH = _N // _CH
_CHV = _CH // 16


def _stage_a(no_ref, t_ref, mp_ref, res_ref, mt_ref, mpx_ref):
    no = no_ref[0]
    t = t_ref[0]
    res_ref[0] = jnp.maximum(-(t * jnp.log(no)), 0.0)
    mt_ref[...] = jnp.full((1, 1, 128), jnp.max(t), jnp.float32)
    mpx_ref[...] = jnp.full((1, 1, 128), jnp.max(mp_ref[0]), jnp.float32)


_SC_MESH = plsc.VectorSubcoreMesh(core_axis_name="c", subcore_axis_name="s")


@functools.partial(
    pl.kernel,
    out_type=jax.ShapeDtypeStruct((_NSLICES, 16), jnp.float32),
    mesh=_SC_MESH,
    scratch_types=[
        pltpu.VMEM((_CH,), jnp.float32),
        pltpu.VMEM((2048,), jnp.float32),
        pltpu.VMEM((2048,), jnp.float32),
        pltpu.VMEM((16,), jnp.float32),
    ],
    compiler_params=pltpu.CompilerParams(needs_layout_passes=False),
)
def _sc_topk(res_hbm, out_hbm, buf, hcnt, hsum, loss_ref):
    c = lax.axis_index("c")
    s = lax.axis_index("s")
    wid = c * 16 + s

    @pl.when(wid < _NSLICES)
    def _():
        ones = jnp.ones((16,), jnp.float32)
        io16 = lax.broadcasted_iota(jnp.int32, (16,), 0)

        def hist_round(sh, msk, ash, prefix):
            def zbody(i, carry):
                z16 = jnp.zeros((16,), jnp.float32)
                hcnt[pl.ds(i * 16, 16)] = z16
                hsum[pl.ds(i * 16, 16)] = z16
                return carry

            lax.fori_loop(0, 2048 // 16, zbody, 0)

            def chunk_body(ch, carry):
                pltpu.sync_copy(res_hbm.at[wid, pl.ds(ch * _CH, _CH)], buf)

                def vreg_body(j, carry2):
                    v = buf[pl.ds(j * 16, 16)]
                    b32 = lax.bitcast_convert_type(v, jnp.int32)
                    dig = lax.shift_right_logical(b32, sh)
                    if msk is not None:
                        dig = jnp.bitwise_and(dig, msk)
                    if ash is None:
                        plsc.addupdate_scatter(hcnt, [dig], ones)
                        plsc.addupdate_scatter(hsum, [dig], v)
                    else:
                        act = lax.shift_right_logical(b32, ash) == prefix
                        plsc.addupdate_scatter(hcnt, [dig], ones, mask=act)
                        plsc.addupdate_scatter(hsum, [dig], v, mask=act)
                    return carry2

                return lax.fori_loop(0, _CHV, vreg_body, carry)

            lax.fori_loop(0, _NCH, chunk_body, 0)

        def scan_round(nbins, kleft):
            nv = nbins // 16

            def sbody(i, carry):
                done, bsel, cex, sex, cnt_ab, sum_ab = carry
                idx = nv - 1 - i
                cv = hcnt[pl.ds(idx * 16, 16)]
                sv = hsum[pl.ds(idx * 16, 16)]
                tot_c = jnp.sum(cv)
                tot_s = jnp.sum(sv)
                suf = lax.rev(plsc.cumsum(lax.rev(cv, (0,))), (0,))
                m = (cnt_ab + suf) >= kleft
                np_s = jnp.max(plsc.all_reduce_population_count(m))
                found = jnp.logical_and(jnp.logical_not(done), np_s >= 1)
                b_l = np_s - 1
                above = io16 > b_l
                c_excl = cnt_ab + jnp.sum(jnp.where(above, cv, 0.0))
                s_excl = sum_ab + jnp.sum(jnp.where(above, sv, 0.0))
                bsel = jnp.where(found, idx * 16 + b_l, bsel)
                cex = jnp.where(found, c_excl, cex)
                sex = jnp.where(found, s_excl, sex)
                done = jnp.logical_or(done, found)
                return (done, bsel, cex, sex, cnt_ab + tot_c, sum_ab + tot_s)

            init = (jnp.bool_(False), jnp.int32(0), jnp.float32(0.0),
                    jnp.float32(0.0), jnp.float32(0.0), jnp.float32(0.0))
            _, bsel, cex, sex, _, _ = lax.fori_loop(0, nv, sbody, init)
            return bsel, cex, sex

        kleft = jnp.float32(_K)
        sum_above = jnp.float32(0.0)

        hist_round(21, None, None, None)
        b1, cex, sex = scan_round(1024, kleft)
        kleft = kleft - cex
        sum_above = sum_above + sex

        hist_round(10, 2047, 21, b1)
        b2, cex, sex = scan_round(2048, kleft)
        kleft = kleft - cex
        sum_above = sum_above + sex
        p2 = lax.shift_left(b1, 11) | b2

        hist_round(0, 1023, 10, p2)
        b3, cex, sex = scan_round(1024, kleft)
        kleft = kleft - cex
        sum_above = sum_above + sex

        pivot_bits = lax.shift_left(p2, 10) | b3
        pv = jnp.max(lax.bitcast_convert_type(
            jnp.full((16,), pivot_bits, jnp.int32), jnp.float32))
        loss = (sum_above + kleft * pv) * jnp.float32(1.0 / _K)
        loss_ref[...] = jnp.full((16,), loss, jnp.float32)
        pltpu.sync_copy(loss_ref, out_hbm.at[wid])


@jax.jit
def kernel(net_out, target, max_positiones):
    shape3 = (_NSLICES, _ROWS, 128)
    no = net_out.reshape(shape3)
    t = target.reshape(shape3)
    mp = max_positiones.reshape(shape3)
    in_spec = pl.BlockSpec((1, _ROWS, 128), lambda i: (i, 0, 0))
    res, mt, mpx = pl.pallas_call(
        _stage_a,
        grid=(_NSLICES,),
        in_specs=[in_spec, in_spec, in_spec],
        out_specs=[
            pl.BlockSpec((1, _ROWS, 128), lambda i: (i, 0, 0)),
            pl.BlockSpec((1, 1, 128), lambda i: (i, 0, 0)),
            pl.BlockSpec((1, 1, 128), lambda i: (i, 0, 0)),
        ],
        out_shape=[
            jax.ShapeDtypeStruct(shape3, jnp.float32),
            jax.ShapeDtypeStruct((_NSLICES, 1, 128), jnp.float32),
            jax.ShapeDtypeStruct((_NSLICES, 1, 128), jnp.float32),
        ],
    )(no, t, mp)
    sc_loss = _sc_topk(res.reshape(_NSLICES, _N))[:, 0]
    skip = (mt[:, 0, 0] == 0.0) & (mpx[:, 0, 0] == 0.0)
    per = jnp.where(skip, 0.0, sc_loss).reshape(_B, _C)
    counts = jnp.count_nonzero(per, axis=1)
    img_losses = per.sum(axis=1) / counts
    return img_losses.sum() / _B


# speedup vs baseline: 3.5822x; 3.5822x over previous
import functools

import jax
import jax.numpy as jnp
from jax import lax
from jax.experimental import pallas as pl
from jax.experimental.pallas import tpu as pltpu
from jax.experimental.pallas import tpu_sc as plsc

_B, _C, _H, _W = 4, 4, 384, 384
_N = _H * _W
_K = _N * 10 // 100
_ROWS = _N // 128
_NSLICES = _B * _C
_HALF = _N // 2
_HALFV = _HALF // 16


def _stage_a(no_ref, t_ref, mp_ref, res_ref, mt_ref, mpx_ref):
    no = no_ref[0]
    t = t_ref[0]
    res_ref[0] = jnp.maximum(-(t * jnp.log(no)), 0.0)
    mt_ref[...] = jnp.full((1, 1, 128), jnp.max(t), jnp.float32)
    mpx_ref[...] = jnp.full((1, 1, 128), jnp.max(mp_ref[0]), jnp.float32)


_SC_MESH = plsc.VectorSubcoreMesh(core_axis_name="c", subcore_axis_name="s")


@functools.partial(
    pl.kernel,
    out_type=jax.ShapeDtypeStruct((_NSLICES, 16), jnp.float32),
    mesh=_SC_MESH,
    scratch_types=[
        pltpu.VMEM((_HALF,), jnp.float32),
        pltpu.VMEM((2048,), jnp.float32),
        pltpu.VMEM((2048,), jnp.float32),
        pltpu.VMEM((4096,), jnp.float32),
        pltpu.VMEM((16,), jnp.float32),
        pltpu.VMEM_SHARED((16, 4096), jnp.float32),
    ],
    compiler_params=pltpu.CompilerParams(needs_layout_passes=False),
)
def _sc_topk(res_hbm, out_hbm, data, hcnt, hsum, pbuf, loss_ref, shared):
    c = lax.axis_index("c")
    s = lax.axis_index("s")
    sl = c * 8 + lax.shift_right_logical(s, 1)
    half = jnp.bitwise_and(s, 1)
    partner = jnp.bitwise_xor(s, 1)

    pltpu.sync_copy(res_hbm.at[sl, pl.ds(half * _HALF, _HALF)], data)

    if True:
        ones = jnp.ones((16,), jnp.float32)
        io16 = lax.broadcasted_iota(jnp.int32, (16,), 0)

        def hist_round(sh, msk, ash, prefix):
            @functools.partial(plsc.parallel_loop, 0, 2048 // 16, unroll=8)
            def _z(i):
                z16 = jnp.zeros((16,), jnp.float32)
                hcnt[pl.ds(i * 16, 16)] = z16
                hsum[pl.ds(i * 16, 16)] = z16

            @functools.partial(plsc.parallel_loop, 0, _HALFV, unroll=8)
            def _h(j):
                v = data[pl.ds(j * 16, 16)]
                b32 = lax.bitcast_convert_type(v, jnp.int32)
                dig = lax.shift_right_logical(b32, sh)
                if msk is not None:
                    dig = jnp.bitwise_and(dig, msk)
                if ash is None:
                    plsc.addupdate_scatter(hcnt, [dig], ones)
                    plsc.addupdate_scatter(hsum, [dig], v)
                else:
                    act = lax.shift_right_logical(b32, ash) == prefix
                    plsc.addupdate_scatter(hcnt, [dig], ones, mask=act)
                    plsc.addupdate_scatter(hsum, [dig], v, mask=act)

            pltpu.sync_copy(hcnt, shared.at[s, pl.ds(0, 2048)])
            pltpu.sync_copy(hsum, shared.at[s, pl.ds(2048, 2048)])
            plsc.subcore_barrier()
            pltpu.sync_copy(shared.at[partner], pbuf)
            plsc.subcore_barrier()

            @functools.partial(plsc.parallel_loop, 0, 2048 // 16, unroll=8)
            def _m(i):
                hcnt[pl.ds(i * 16, 16)] += pbuf[pl.ds(i * 16, 16)]
                hsum[pl.ds(i * 16, 16)] += pbuf[pl.ds(2048 + i * 16, 16)]

        def scan_round(nbins, kleft):
            nv = nbins // 16

            def sbody(i, carry):
                done, bsel, cex, sex, cnt_ab, sum_ab = carry
                idx = nv - 1 - i
                cv = hcnt[pl.ds(idx * 16, 16)]
                sv = hsum[pl.ds(idx * 16, 16)]
                tot_c = jnp.sum(cv)
                tot_s = jnp.sum(sv)
                suf = lax.rev(plsc.cumsum(lax.rev(cv, (0,))), (0,))
                m = (cnt_ab + suf) >= kleft
                np_s = jnp.max(plsc.all_reduce_population_count(m))
                found = jnp.logical_and(jnp.logical_not(done), np_s >= 1)
                b_l = np_s - 1
                above = io16 > b_l
                c_excl = cnt_ab + jnp.sum(jnp.where(above, cv, 0.0))
                s_excl = sum_ab + jnp.sum(jnp.where(above, sv, 0.0))
                bsel = jnp.where(found, idx * 16 + b_l, bsel)
                cex = jnp.where(found, c_excl, cex)
                sex = jnp.where(found, s_excl, sex)
                done = jnp.logical_or(done, found)
                return (done, bsel, cex, sex, cnt_ab + tot_c, sum_ab + tot_s)

            init = (jnp.bool_(False), jnp.int32(0), jnp.float32(0.0),
                    jnp.float32(0.0), jnp.float32(0.0), jnp.float32(0.0))
            _, bsel, cex, sex, _, _ = lax.fori_loop(0, nv, sbody, init)
            return bsel, cex, sex

        kleft = jnp.float32(_K)
        sum_above = jnp.float32(0.0)

        hist_round(21, None, None, None)
        b1, cex, sex = scan_round(1024, kleft)
        kleft = kleft - cex
        sum_above = sum_above + sex

        hist_round(10, 2047, 21, b1)
        b2, cex, sex = scan_round(2048, kleft)
        kleft = kleft - cex
        sum_above = sum_above + sex
        p2 = lax.shift_left(b1, 11) | b2

        hist_round(0, 1023, 10, p2)
        b3, cex, sex = scan_round(1024, kleft)
        kleft = kleft - cex
        sum_above = sum_above + sex

        pivot_bits = lax.shift_left(p2, 10) | b3
        pv = jnp.max(lax.bitcast_convert_type(
            jnp.full((16,), pivot_bits, jnp.int32), jnp.float32))
        loss = (sum_above + kleft * pv) * jnp.float32(1.0 / _K)

        @pl.when(half == 0)
        def _():
            loss_ref[...] = jnp.full((16,), loss, jnp.float32)
            pltpu.sync_copy(loss_ref, out_hbm.at[sl])


@jax.jit
def kernel(net_out, target, max_positiones):
    shape3 = (_NSLICES, _ROWS, 128)
    no = net_out.reshape(shape3)
    t = target.reshape(shape3)
    mp = max_positiones.reshape(shape3)
    in_spec = pl.BlockSpec((1, _ROWS, 128), lambda i: (i, 0, 0))
    res, mt, mpx = pl.pallas_call(
        _stage_a,
        grid=(_NSLICES,),
        in_specs=[in_spec, in_spec, in_spec],
        out_specs=[
            pl.BlockSpec((1, _ROWS, 128), lambda i: (i, 0, 0)),
            pl.BlockSpec((1, 1, 128), lambda i: (i, 0, 0)),
            pl.BlockSpec((1, 1, 128), lambda i: (i, 0, 0)),
        ],
        out_shape=[
            jax.ShapeDtypeStruct(shape3, jnp.float32),
            jax.ShapeDtypeStruct((_NSLICES, 1, 128), jnp.float32),
            jax.ShapeDtypeStruct((_NSLICES, 1, 128), jnp.float32),
        ],
    )(no, t, mp)
    sc_loss = _sc_topk(res.reshape(_NSLICES, _N))[:, 0]
    skip = (mt[:, 0, 0] == 0.0) & (mpx[:, 0, 0] == 0.0)
    per = jnp.where(skip, 0.0, sc_loss).reshape(_B, _C)
    counts = jnp.count_nonzero(per, axis=1)
    img_losses = per.sum(axis=1) / counts
    return img_losses.sum() / _B


# speedup vs baseline: 6.3832x; 1.7819x over previous
import functools

import jax
import jax.numpy as jnp
from jax import lax
from jax.experimental import pallas as pl
from jax.experimental.pallas import tpu as pltpu
from jax.experimental.pallas import tpu_sc as plsc

_B, _C, _H, _W = 4, 4, 384, 384
_N = _H * _W
_K = _N * 10 // 100
_ROWS = _N // 128
_NSLICES = _B * _C
_HALF = _N // 2
_HALFV = _HALF // 16


def _stage_a(no_ref, t_ref, mp_ref, res_ref, mt_ref, mpx_ref):
    no = no_ref[0, 0]
    t = t_ref[0, 0]
    res_ref[0, 0] = jnp.maximum(-(t * jnp.log(no)), 0.0)
    mt_ref[...] = jnp.full((1, 1, 1, 128), jnp.max(t), jnp.float32)
    mpx_ref[...] = jnp.full((1, 1, 1, 128), jnp.max(mp_ref[0, 0]), jnp.float32)


_SC_MESH = plsc.VectorSubcoreMesh(core_axis_name="c", subcore_axis_name="s")


@functools.partial(
    pl.kernel,
    out_type=jax.ShapeDtypeStruct((_NSLICES, 16), jnp.float32),
    mesh=_SC_MESH,
    scratch_types=[
        pltpu.VMEM((_H // 2, _W), jnp.float32),
        pltpu.VMEM((2048,), jnp.float32),
        pltpu.VMEM((2048,), jnp.float32),
        pltpu.VMEM((4096,), jnp.float32),
        pltpu.VMEM((16,), jnp.float32),
        pltpu.VMEM_SHARED((16, 4096), jnp.float32),
    ],
    compiler_params=pltpu.CompilerParams(needs_layout_passes=False),
)
def _sc_topk(res_hbm, out_hbm, data, hcnt, hsum, pbuf, loss_ref, shared):
    c = lax.axis_index("c")
    s = lax.axis_index("s")
    sl = c * 8 + lax.shift_right_logical(s, 1)
    half = jnp.bitwise_and(s, 1)
    partner = jnp.bitwise_xor(s, 1)

    pltpu.sync_copy(
        res_hbm.at[lax.shift_right_logical(sl, 2), jnp.bitwise_and(sl, 3),
                   pl.ds(half * (_H // 2), _H // 2)],
        data)

    if True:
        ones = jnp.ones((16,), jnp.float32)
        io16 = lax.broadcasted_iota(jnp.int32, (16,), 0)

        def hist_round(sh, msk, ash, prefix):
            @functools.partial(plsc.parallel_loop, 0, 2048 // 16, unroll=8)
            def _z(i):
                z16 = jnp.zeros((16,), jnp.float32)
                hcnt[pl.ds(i * 16, 16)] = z16
                hsum[pl.ds(i * 16, 16)] = z16

            @functools.partial(plsc.parallel_loop, 0, _H // 2, unroll=2)
            def _h(r):
                for k in range(_W // 16):
                    v = data[r, pl.ds(k * 16, 16)]
                    b32 = lax.bitcast_convert_type(v, jnp.int32)
                    dig = lax.shift_right_logical(b32, sh)
                    if msk is not None:
                        dig = jnp.bitwise_and(dig, msk)
                    if ash is None:
                        plsc.addupdate_scatter(hcnt, [dig], ones)
                        plsc.addupdate_scatter(hsum, [dig], v)
                    else:
                        act = lax.shift_right_logical(b32, ash) == prefix
                        plsc.addupdate_scatter(hcnt, [dig], ones, mask=act)
                        plsc.addupdate_scatter(hsum, [dig], v, mask=act)

            pltpu.sync_copy(hcnt, shared.at[s, pl.ds(0, 2048)])
            pltpu.sync_copy(hsum, shared.at[s, pl.ds(2048, 2048)])
            plsc.subcore_barrier()
            pltpu.sync_copy(shared.at[partner], pbuf)
            plsc.subcore_barrier()

            @functools.partial(plsc.parallel_loop, 0, 2048 // 16, unroll=8)
            def _m(i):
                hcnt[pl.ds(i * 16, 16)] += pbuf[pl.ds(i * 16, 16)]
                hsum[pl.ds(i * 16, 16)] += pbuf[pl.ds(2048 + i * 16, 16)]

        def scan_round(nbins, kleft):
            nv = nbins // 16

            def sbody(i, carry):
                done, bsel, cex, sex, cnt_ab, sum_ab = carry
                idx = nv - 1 - i
                cv = hcnt[pl.ds(idx * 16, 16)]
                sv = hsum[pl.ds(idx * 16, 16)]
                tot_c = jnp.sum(cv)
                tot_s = jnp.sum(sv)
                suf = lax.rev(plsc.cumsum(lax.rev(cv, (0,))), (0,))
                m = (cnt_ab + suf) >= kleft
                np_s = jnp.max(plsc.all_reduce_population_count(m))
                found = jnp.logical_and(jnp.logical_not(done), np_s >= 1)
                b_l = np_s - 1
                above = io16 > b_l
                c_excl = cnt_ab + jnp.sum(jnp.where(above, cv, 0.0))
                s_excl = sum_ab + jnp.sum(jnp.where(above, sv, 0.0))
                bsel = jnp.where(found, idx * 16 + b_l, bsel)
                cex = jnp.where(found, c_excl, cex)
                sex = jnp.where(found, s_excl, sex)
                done = jnp.logical_or(done, found)
                return (done, bsel, cex, sex, cnt_ab + tot_c, sum_ab + tot_s)

            init = (jnp.bool_(False), jnp.int32(0), jnp.float32(0.0),
                    jnp.float32(0.0), jnp.float32(0.0), jnp.float32(0.0))
            _, bsel, cex, sex, _, _ = lax.fori_loop(0, nv, sbody, init)
            return bsel, cex, sex

        kleft = jnp.float32(_K)
        sum_above = jnp.float32(0.0)

        hist_round(21, None, None, None)
        b1, cex, sex = scan_round(1024, kleft)
        kleft = kleft - cex
        sum_above = sum_above + sex

        hist_round(10, 2047, 21, b1)
        b2, cex, sex = scan_round(2048, kleft)
        kleft = kleft - cex
        sum_above = sum_above + sex
        p2 = lax.shift_left(b1, 11) | b2

        hist_round(0, 1023, 10, p2)
        b3, cex, sex = scan_round(1024, kleft)
        kleft = kleft - cex
        sum_above = sum_above + sex

        pivot_bits = lax.shift_left(p2, 10) | b3
        pv = jnp.max(lax.bitcast_convert_type(
            jnp.full((16,), pivot_bits, jnp.int32), jnp.float32))
        loss = (sum_above + kleft * pv) * jnp.float32(1.0 / _K)

        @pl.when(half == 0)
        def _():
            loss_ref[...] = jnp.full((16,), loss, jnp.float32)
            pltpu.sync_copy(loss_ref, out_hbm.at[sl])


@jax.jit
def kernel(net_out, target, max_positiones):
    in_spec = pl.BlockSpec((1, 1, _H, _W), lambda i, j: (i, j, 0, 0))
    res, mt, mpx = pl.pallas_call(
        _stage_a,
        grid=(_B, _C),
        in_specs=[in_spec, in_spec, in_spec],
        out_specs=[
            pl.BlockSpec((1, 1, _H, _W), lambda i, j: (i, j, 0, 0)),
            pl.BlockSpec((1, 1, 1, 128), lambda i, j: (i, j, 0, 0)),
            pl.BlockSpec((1, 1, 1, 128), lambda i, j: (i, j, 0, 0)),
        ],
        out_shape=[
            jax.ShapeDtypeStruct((_B, _C, _H, _W), jnp.float32),
            jax.ShapeDtypeStruct((_B, _C, 1, 128), jnp.float32),
            jax.ShapeDtypeStruct((_B, _C, 1, 128), jnp.float32),
        ],
    )(net_out, target, max_positiones)
    sc_loss = _sc_topk(res)[:, 0].reshape(_B, _C)
    skip = (mt[:, :, 0, 0] == 0.0) & (mpx[:, :, 0, 0] == 0.0)
    per = jnp.where(skip, 0.0, sc_loss)
    counts = jnp.count_nonzero(per, axis=1)
    img_losses = per.sum(axis=1) / counts
    return img_losses.sum() / _B
